# Initial kernel scaffold; baseline (speedup 1.0000x reference)
import functools

import jax
import jax.numpy as jnp
from jax import lax
from jax.experimental import pallas as pl
from jax.experimental.pallas import tpu as pltpu
from jax.experimental.pallas import tpu_sc as plsc

NUM_LABELS = 1000
B = 32
SEQ = 128
HID = 768
DMODEL = 1024
TYPE_DIM = DMODEL - HID
E_LABEL = 4000

NC = 2
NS = 16
NW = NC * NS
LANES = 16

_CH = 32
_EPW = 128
_FULL_W = E_LABEL // _EPW
_REM_CHUNKS = (E_LABEL - _FULL_W * _EPW) // _CH
_NTOKC = SEQ // _CH
_AGG_ROWS = 1024
_RPT = _AGG_ROWS // NS


def _sc_body(ids_hbm, tok_hbm, lab_hbm, src_hbm, dst_hbm,
             text_out, seg_out,
             tok_idx, tok_buf, acc, eidx, erow, agg_sh, sem):
    c = lax.axis_index("c")
    s = lax.axis_index("s")
    wid = s * NC + c
    zero16 = jnp.zeros((LANES,), jnp.float32)

    def _zrow(j, carry):
        for t in range(DMODEL // LANES):
            erow[j, pl.ds(t * LANES, LANES)] = zero16
        return carry
    lax.fori_loop(0, _CH, _zrow, 0)
    for r in range(_RPT // _CH):
        pltpu.sync_copy(erow, agg_sh.at[pl.ds(s * _RPT + r * _CH, _CH)])
    plsc.subcore_barrier()

    pltpu.sync_copy(ids_hbm.at[wid], tok_idx)
    for t in range(HID // LANES):
        acc[pl.ds(t * LANES, LANES)] = zero16
    for ckn in range(_NTOKC):
        pltpu.async_copy(
            tok_hbm.at[tok_idx.at[pl.ds(ckn * _CH, _CH)]], tok_buf, sem
        ).wait()

        def _accum(j, carry):
            for t in range(HID // LANES):
                sl = pl.ds(t * LANES, LANES)
                acc[sl] = acc[sl] + tok_buf[j, sl]
            return carry
        lax.fori_loop(0, _CH, _accum, 0)
    inv = jnp.float32(1.0 / SEQ)
    for t in range(HID // LANES):
        sl = pl.ds(t * LANES, LANES)
        acc[sl] = acc[sl] * inv
    pltpu.sync_copy(acc, text_out.at[wid])

    base = wid * _EPW
    nchunks = jnp.where(wid < _FULL_W, _EPW // _CH, _REM_CHUNKS)

    def _echunk(ckn, carry):
        off = base + ckn * _CH
        pltpu.sync_copy(src_hbm.at[pl.ds(off, _CH)], eidx.at[0])
        pltpu.sync_copy(dst_hbm.at[pl.ds(off, _CH)], eidx.at[1])
        pltpu.async_copy(lab_hbm.at[eidx.at[0]], erow, sem).wait()
        pltpu.sync_copy(erow, agg_sh.at[eidx.at[1]], add=True)
        return carry
    lax.fori_loop(0, nchunks, _echunk, 0)

    plsc.subcore_barrier()
    pltpu.sync_copy(agg_sh.at[pl.ds(s * _RPT, _RPT)],
                    seg_out.at[c, pl.ds(s * _RPT, _RPT)])


@functools.cache
def _make_sc(interpret=False):
    return pl.kernel(
        _sc_body,
        out_type=(
            jax.ShapeDtypeStruct((B, HID), jnp.float32),
            jax.ShapeDtypeStruct((NC, _AGG_ROWS, DMODEL), jnp.float32),
        ),
        mesh=plsc.VectorSubcoreMesh(core_axis_name="c", subcore_axis_name="s"),
        scratch_types=[
            pltpu.VMEM((SEQ,), jnp.int32),
            pltpu.VMEM((_CH, HID), jnp.float32),
            pltpu.VMEM((HID,), jnp.float32),
            pltpu.VMEM((2, _CH), jnp.int32),
            pltpu.VMEM((_CH, DMODEL), jnp.float32),
            pltpu.VMEM_SHARED((_AGG_ROWS, DMODEL), jnp.float32),
            pltpu.SemaphoreType.DMA,
        ],
        interpret=interpret,
    )


def _tc_body(text_ref, seg_ref, lab_ref, nte_ref, w1_ref, b1_ref, w2_ref,
             b2_ref, out_ref):
    text = text_ref[...]
    nte1 = nte_ref[1, :]
    text_nodes = jnp.concatenate(
        [text, jnp.broadcast_to(nte1[None, :], (B, TYPE_DIM))], axis=1)
    s_text = jnp.sum(text_nodes, axis=0)
    s_lab = jnp.sum(lab_ref[...], axis=0)
    seg = seg_ref[0, :NUM_LABELS, :] + seg_ref[1, :NUM_LABELS, :]
    agg = jnp.concatenate(
        [seg + s_text[None, :], text_nodes + s_lab[None, :]], axis=0)
    h = jnp.maximum(
        jnp.dot(agg, w1_ref[...], preferred_element_type=jnp.float32)
        + b1_ref[...][None, :], 0.0)
    out_ref[...] = (
        jnp.dot(h, w2_ref[...], preferred_element_type=jnp.float32)
        + b2_ref[...][None, :])


@functools.cache
def _make_tc(interpret=False):
    return pl.pallas_call(
        _tc_body,
        out_shape=jax.ShapeDtypeStruct((NUM_LABELS + B, NUM_LABELS),
                                       jnp.float32),
        interpret=interpret,
    )


def kernel(input_ids, token_table, node_type_embeddings, label_nodes,
           label_edges, W1, b1, W2, b2):
    src = label_edges[0]
    dst = label_edges[1]
    text_emb, seg = _make_sc()(input_ids, token_table, label_nodes, src, dst)
    return _make_tc()(text_emb, seg, label_nodes, node_type_embeddings,
                      W1, b1, W2, b2)


# speedup vs baseline: 13.2293x; 13.2293x over previous
import functools

import jax
import jax.numpy as jnp
from jax import lax
from jax.experimental import pallas as pl
from jax.experimental.pallas import tpu as pltpu
from jax.experimental.pallas import tpu_sc as plsc

NUM_LABELS = 1000
B = 32
SEQ = 128
HID = 768
DMODEL = 1024
TYPE_DIM = DMODEL - HID
E_LABEL = 4000

NC = 2
NS = 16
NW = NC * NS
LANES = 16

_TCH = 32
_NTCH = SEQ // _TCH
_CW = DMODEL // NW
_SRC_ROWS = NW * _CW
_DCOL = 1008
_NEG = E_LABEL // LANES


def _sc_body(ids_hbm, tok_hbm, src_hbm, dst_hbm,
             text_out, mt_out,
             tcidx, tok_buf, acc, esrc, edst, mt_loc, sem):
    c = lax.axis_index("c")
    s = lax.axis_index("s")
    wid = s * NC + c
    zero16 = jnp.zeros((LANES,), jnp.float32)

    for t in range(HID // LANES):
        acc[pl.ds(t * LANES, LANES)] = zero16
    for ckn in range(_NTCH):
        pltpu.sync_copy(ids_hbm.at[wid, pl.ds(ckn * _TCH, _TCH)], tcidx)
        pltpu.async_copy(tok_hbm.at[tcidx], tok_buf, sem).wait()

        def _accum(j, carry):
            for t in range(HID // LANES):
                sl = pl.ds(t * LANES, LANES)
                acc[sl] = acc[sl] + tok_buf[j, sl]
            return carry
        lax.fori_loop(0, _TCH, _accum, 0)
    inv = jnp.float32(1.0 / SEQ)
    for t in range(HID // LANES):
        sl = pl.ds(t * LANES, LANES)
        acc[sl] = acc[sl] * inv
    pltpu.sync_copy(acc, text_out.at[wid])

    def _zrow(j, carry):
        for t in range(_DCOL // LANES):
            mt_loc[j, pl.ds(t * LANES, LANES)] = zero16
        return carry
    lax.fori_loop(0, _CW, _zrow, 0)

    pltpu.sync_copy(src_hbm, esrc)
    pltpu.sync_copy(dst_hbm, edst)
    iota16 = lax.iota(jnp.int32, LANES)
    ones16 = jnp.ones((LANES,), jnp.float32)

    widv = lax.broadcast(wid, (LANES,))
    cwv = jnp.full((LANES,), _CW, jnp.int32)
    lanesv = jnp.full((LANES,), LANES, jnp.int32)

    def _egroup(g, carry):
        gof = g * LANES
        sv = esrc[pl.ds(gof, LANES)]
        dv = edst[pl.ds(gof, LANES)]
        sblk = lax.div(sv, cwv)
        slv = lax.rem(sv, cwv)
        val = jnp.where(sblk == widv, ones16, zero16)
        col_base = lax.mul(lax.div(dv, lanesv), lanesv)
        col_in = lax.sub(dv, col_base)
        for i in range(LANES):
            row = slv[i]
            sl = pl.ds(pl.multiple_of(col_base[i], LANES), LANES)
            addv = jnp.where(iota16 == lax.broadcast(col_in[i], (LANES,)),
                             lax.broadcast(val[i], (LANES,)), zero16)
            mt_loc[row, sl] = mt_loc[row, sl] + addv
        return carry
    lax.fori_loop(0, _NEG, _egroup, 0)

    pltpu.sync_copy(mt_loc, mt_out.at[pl.ds(wid * _CW, _CW)])


@functools.cache
def _make_sc(interpret=False):
    return pl.kernel(
        _sc_body,
        out_type=(
            jax.ShapeDtypeStruct((B, HID), jnp.float32),
            jax.ShapeDtypeStruct((_SRC_ROWS, _DCOL), jnp.float32),
        ),
        mesh=plsc.VectorSubcoreMesh(core_axis_name="c", subcore_axis_name="s",
                                    num_cores=NC, num_subcores=NS),
        scratch_types=[
            pltpu.VMEM((_TCH,), jnp.int32),
            pltpu.VMEM((_TCH, HID), jnp.float32),
            pltpu.VMEM((HID,), jnp.float32),
            pltpu.VMEM((E_LABEL,), jnp.int32),
            pltpu.VMEM((E_LABEL,), jnp.int32),
            pltpu.VMEM((_CW, _DCOL), jnp.float32),
            pltpu.SemaphoreType.DMA,
        ],
        interpret=interpret,
    )


def _tc_body(text_ref, mt_ref, lab_ref, nte_ref, w1_ref, b1_ref, w2_ref,
             b2_ref, out_ref):
    text = text_ref[...]
    nte1 = nte_ref[1, :]
    text_nodes = jnp.concatenate(
        [text, jnp.broadcast_to(nte1[None, :], (B, TYPE_DIM))], axis=1)
    s_text = jnp.sum(text_nodes, axis=0)
    lab = lab_ref[...]
    s_lab = jnp.sum(lab, axis=0)
    mt = mt_ref[...][:NUM_LABELS, :]
    seg_t = lax.dot_general(mt, lab, (((0,), (0,)), ((), ())),
                            preferred_element_type=jnp.float32)
    seg = seg_t[:NUM_LABELS, :]
    agg = jnp.concatenate(
        [seg + s_text[None, :], text_nodes + s_lab[None, :]], axis=0)
    h = jnp.maximum(
        jnp.dot(agg, w1_ref[...], preferred_element_type=jnp.float32)
        + b1_ref[...][None, :], 0.0)
    out_ref[...] = (
        jnp.dot(h, w2_ref[...], preferred_element_type=jnp.float32)
        + b2_ref[...][None, :])


@functools.cache
def _make_tc(interpret=False):
    return pl.pallas_call(
        _tc_body,
        out_shape=jax.ShapeDtypeStruct((NUM_LABELS + B, NUM_LABELS),
                                       jnp.float32),
        interpret=interpret,
    )


def kernel(input_ids, token_table, node_type_embeddings, label_nodes,
           label_edges, W1, b1, W2, b2):
    src = label_edges[0]
    dst = label_edges[1]
    text_emb, mt = _make_sc()(input_ids, token_table, src, dst)
    return _make_tc()(text_emb, mt, label_nodes, node_type_embeddings,
                      W1, b1, W2, b2)


# speedup vs baseline: 24.0247x; 1.8160x over previous
import functools

import jax
import jax.numpy as jnp
from jax import lax
from jax.experimental import pallas as pl
from jax.experimental.pallas import tpu as pltpu
from jax.experimental.pallas import tpu_sc as plsc

NUM_LABELS = 1000
B = 32
SEQ = 128
HID = 768
DMODEL = 1024
TYPE_DIM = DMODEL - HID
E_LABEL = 4000

NC = 2
NS = 16
NW = NC * NS
LANES = 16

_TCH = 32
_CW = DMODEL // NW
_SRC_ROWS = NW * _CW
_DCOL = 1008
_NEG = E_LABEL // LANES


def _sc_body(ids_hbm, tok_hbm, src_hbm, dst_hbm,
             text_out, mt_out,
             tcidx_a, tcidx_b, buf_a, buf_b, acc, esrc, edst, mt_loc,
             sem_a, sem_b):
    c = lax.axis_index("c")
    s = lax.axis_index("s")
    wid = s * NC + c
    zero16 = jnp.zeros((LANES,), jnp.float32)

    pltpu.sync_copy(ids_hbm.at[wid, pl.ds(0 * _TCH, _TCH)], tcidx_a)
    cp_a = pltpu.async_copy(tok_hbm.at[tcidx_a], buf_a, sem_a)
    pltpu.sync_copy(ids_hbm.at[wid, pl.ds(1 * _TCH, _TCH)], tcidx_b)
    cp_b = pltpu.async_copy(tok_hbm.at[tcidx_b], buf_b, sem_b)
    cp_a.wait()
    pltpu.sync_copy(ids_hbm.at[wid, pl.ds(2 * _TCH, _TCH)], tcidx_a)
    cp_a2 = pltpu.async_copy(tok_hbm.at[tcidx_a], buf_a, sem_a, add=True)
    cp_b.wait()
    pltpu.sync_copy(ids_hbm.at[wid, pl.ds(3 * _TCH, _TCH)], tcidx_b)
    cp_b2 = pltpu.async_copy(tok_hbm.at[tcidx_b], buf_b, sem_b, add=True)
    cp_a2.wait()
    cp_b2.wait()

    inv = jnp.float32(1.0 / SEQ)
    for t in range(HID // LANES):
        sl = pl.ds(t * LANES, LANES)

        def _red(j, r):
            return r + buf_a[j, sl] + buf_b[j, sl]
        acc[sl] = lax.fori_loop(0, _TCH, _red, zero16) * inv
    pltpu.sync_copy(acc, text_out.at[wid])

    def _zrow(j, carry):
        for t in range(_DCOL // LANES):
            mt_loc[j, pl.ds(t * LANES, LANES)] = zero16
        return carry
    lax.fori_loop(0, _CW, _zrow, 0)

    pltpu.sync_copy(src_hbm, esrc)
    pltpu.sync_copy(dst_hbm, edst)
    widv = lax.broadcast(wid, (LANES,))
    cwv = jnp.full((LANES,), _CW, jnp.int32)
    ones16 = jnp.ones((LANES,), jnp.float32)

    def _egroup(g, carry):
        gof = g * LANES
        sv = esrc[pl.ds(gof, LANES)]
        dv = edst[pl.ds(gof, LANES)]
        sblk = lax.div(sv, cwv)
        slv = lax.rem(sv, cwv)
        val = jnp.where(sblk == widv, ones16, zero16)
        plsc.addupdate_scatter(mt_loc, [slv, dv], val)
        return carry
    lax.fori_loop(0, _NEG, _egroup, 0)

    pltpu.sync_copy(mt_loc, mt_out.at[pl.ds(wid * _CW, _CW)])


@functools.cache
def _make_sc(interpret=False):
    return pl.kernel(
        _sc_body,
        out_type=(
            jax.ShapeDtypeStruct((B, HID), jnp.float32),
            jax.ShapeDtypeStruct((_SRC_ROWS, _DCOL), jnp.float32),
        ),
        compiler_params=pltpu.CompilerParams(needs_layout_passes=False),
        mesh=plsc.VectorSubcoreMesh(core_axis_name="c", subcore_axis_name="s",
                                    num_cores=NC, num_subcores=NS),
        scratch_types=[
            pltpu.VMEM((_TCH,), jnp.int32),
            pltpu.VMEM((_TCH,), jnp.int32),
            pltpu.VMEM((_TCH, HID), jnp.float32),
            pltpu.VMEM((_TCH, HID), jnp.float32),
            pltpu.VMEM((HID,), jnp.float32),
            pltpu.VMEM((E_LABEL,), jnp.int32),
            pltpu.VMEM((E_LABEL,), jnp.int32),
            pltpu.VMEM((_CW, _DCOL), jnp.float32),
            pltpu.SemaphoreType.DMA,
            pltpu.SemaphoreType.DMA,
        ],
        interpret=interpret,
    )


def _tc_body(text_ref, mt_ref, lab_ref, nte_ref, w1_ref, b1_ref, w2_ref,
             b2_ref, out_ref):
    text = text_ref[...]
    nte1 = nte_ref[1, :]
    text_nodes = jnp.concatenate(
        [text, jnp.broadcast_to(nte1[None, :], (B, TYPE_DIM))], axis=1)
    s_text = jnp.sum(text_nodes, axis=0)
    lab = lab_ref[...]
    s_lab = jnp.sum(lab, axis=0)
    mt = mt_ref[...][:NUM_LABELS, :]
    seg_t = lax.dot_general(mt, lab, (((0,), (0,)), ((), ())),
                            preferred_element_type=jnp.float32)
    seg = seg_t[:NUM_LABELS, :]
    agg = jnp.concatenate(
        [seg + s_text[None, :], text_nodes + s_lab[None, :]], axis=0)
    h = jnp.maximum(
        jnp.dot(agg, w1_ref[...], preferred_element_type=jnp.float32)
        + b1_ref[...][None, :], 0.0)
    out_ref[...] = (
        jnp.dot(h, w2_ref[...], preferred_element_type=jnp.float32)
        + b2_ref[...][None, :])


@functools.cache
def _make_tc(interpret=False):
    return pl.pallas_call(
        _tc_body,
        out_shape=jax.ShapeDtypeStruct((NUM_LABELS + B, NUM_LABELS),
                                       jnp.float32),
        interpret=interpret,
    )


def kernel(input_ids, token_table, node_type_embeddings, label_nodes,
           label_edges, W1, b1, W2, b2):
    src = label_edges[0]
    dst = label_edges[1]
    text_emb, mt = _make_sc()(input_ids, token_table, src, dst)
    return _make_tc()(text_emb, mt, label_nodes, node_type_embeddings,
                      W1, b1, W2, b2)


# speedup vs baseline: 25.1860x; 1.0483x over previous
import functools

import jax
import jax.numpy as jnp
from jax import lax
from jax.experimental import pallas as pl
from jax.experimental.pallas import tpu as pltpu
from jax.experimental.pallas import tpu_sc as plsc

NUM_LABELS = 1000
B = 32
SEQ = 128
HID = 768
DMODEL = 1024
TYPE_DIM = DMODEL - HID
E_LABEL = 4000

NC = 2
NS = 16
NW = NC * NS
LANES = 16

_TCH = 32
_CW = DMODEL // NW
_SRC_ROWS = NW * _CW
_DCOL = 1008
_NEG = E_LABEL // LANES
_NSLICE = HID // LANES
_RSEG = 16


def _sc_body(ids_hbm, tok_hbm, edges_hbm,
             text_out, mt_out,
             tcidx_a, tcidx_b, buf_a, buf_b, acc, esrc, edst, mt_loc,
             sem_a, sem_b):
    c = lax.axis_index("c")
    s = lax.axis_index("s")
    wid = s * NC + c
    zero16 = jnp.zeros((LANES,), jnp.float32)

    pltpu.sync_copy(ids_hbm.at[wid, pl.ds(0 * _TCH, _TCH)], tcidx_a)
    cp_a = pltpu.async_copy(tok_hbm.at[tcidx_a], buf_a, sem_a)
    pltpu.sync_copy(ids_hbm.at[wid, pl.ds(1 * _TCH, _TCH)], tcidx_b)
    cp_b = pltpu.async_copy(tok_hbm.at[tcidx_b], buf_b, sem_b)

    pltpu.sync_copy(edges_hbm.at[0], esrc)
    pltpu.sync_copy(edges_hbm.at[1], edst)

    def _zrow(j, carry):
        for t in range(_DCOL // LANES):
            mt_loc[j, pl.ds(t * LANES, LANES)] = zero16
        return carry
    lax.fori_loop(0, _CW, _zrow, 0)

    widv = lax.broadcast(wid, (LANES,))
    cwv = jnp.full((LANES,), _CW, jnp.int32)
    ones16 = jnp.ones((LANES,), jnp.float32)

    def _egroup(g, carry):
        gof = g * LANES
        sv = esrc[pl.ds(gof, LANES)]
        sblk = lax.div(sv, cwv)
        mine = sblk == widv
        nhit = plsc.all_reduce_population_count(mine)

        @pl.when(nhit[0] > 0)
        def _():
            dv = edst[pl.ds(gof, LANES)]
            slv = lax.rem(sv, cwv)
            val = jnp.where(mine, ones16, zero16)
            plsc.addupdate_scatter(mt_loc, [slv, dv], val)
        return carry

    lax.fori_loop(0, _NEG // 2, _egroup, 0)
    cp_a.wait()
    pltpu.sync_copy(ids_hbm.at[wid, pl.ds(2 * _TCH, _TCH)], tcidx_a)
    cp_a2 = pltpu.async_copy(tok_hbm.at[tcidx_a], buf_a, sem_a, add=True)
    lax.fori_loop(_NEG // 2, _NEG, _egroup, 0)
    cp_b.wait()
    pltpu.sync_copy(ids_hbm.at[wid, pl.ds(3 * _TCH, _TCH)], tcidx_b)
    cp_b2 = pltpu.async_copy(tok_hbm.at[tcidx_b], buf_b, sem_b, add=True)

    pltpu.sync_copy(mt_loc, mt_out.at[pl.ds(wid * _CW, _CW)])
    cp_a2.wait()
    cp_b2.wait()

    inv = jnp.float32(1.0 / SEQ)
    for blk in range(_NSLICE // _RSEG):
        base = blk * _RSEG * LANES

        def _red(j, regs):
            return tuple(
                regs[t]
                + buf_a[j, pl.ds(base + t * LANES, LANES)]
                + buf_b[j, pl.ds(base + t * LANES, LANES)]
                for t in range(_RSEG))
        regs = lax.fori_loop(0, _TCH, _red, (zero16,) * _RSEG)
        for t in range(_RSEG):
            acc[pl.ds(base + t * LANES, LANES)] = regs[t] * inv
    pltpu.sync_copy(acc, text_out.at[wid])


@functools.cache
def _make_sc(interpret=False):
    return pl.kernel(
        _sc_body,
        out_type=(
            jax.ShapeDtypeStruct((B, HID), jnp.float32),
            jax.ShapeDtypeStruct((_SRC_ROWS, _DCOL), jnp.float32),
        ),
        compiler_params=pltpu.CompilerParams(needs_layout_passes=False),
        mesh=plsc.VectorSubcoreMesh(core_axis_name="c", subcore_axis_name="s",
                                    num_cores=NC, num_subcores=NS),
        scratch_types=[
            pltpu.VMEM((_TCH,), jnp.int32),
            pltpu.VMEM((_TCH,), jnp.int32),
            pltpu.VMEM((_TCH, HID), jnp.float32),
            pltpu.VMEM((_TCH, HID), jnp.float32),
            pltpu.VMEM((HID,), jnp.float32),
            pltpu.VMEM((E_LABEL,), jnp.int32),
            pltpu.VMEM((E_LABEL,), jnp.int32),
            pltpu.VMEM((_CW, _DCOL), jnp.float32),
            pltpu.SemaphoreType.DMA,
            pltpu.SemaphoreType.DMA,
        ],
        interpret=interpret,
    )


def _tc_body(text_ref, mt_ref, lab_ref, nte_ref, w1_ref, b1_ref, w2_ref,
             b2_ref, out_ref):
    text = text_ref[...]
    nte1 = nte_ref[1, :]
    text_nodes = jnp.concatenate(
        [text, jnp.broadcast_to(nte1[None, :], (B, TYPE_DIM))], axis=1)
    s_text = jnp.sum(text_nodes, axis=0)
    lab = lab_ref[...]
    s_lab = jnp.sum(lab, axis=0)
    mt = mt_ref[...][:NUM_LABELS, :]
    seg_t = lax.dot_general(mt, lab, (((0,), (0,)), ((), ())),
                            preferred_element_type=jnp.float32)
    seg = seg_t[:NUM_LABELS, :]
    agg = jnp.concatenate(
        [seg + s_text[None, :], text_nodes + s_lab[None, :]], axis=0)
    h = jnp.maximum(
        jnp.dot(agg, w1_ref[...], preferred_element_type=jnp.float32)
        + b1_ref[...][None, :], 0.0)
    out_ref[...] = (
        jnp.dot(h, w2_ref[...], preferred_element_type=jnp.float32)
        + b2_ref[...][None, :])


@functools.cache
def _make_tc(interpret=False):
    return pl.pallas_call(
        _tc_body,
        out_shape=jax.ShapeDtypeStruct((NUM_LABELS + B, NUM_LABELS),
                                       jnp.float32),
        interpret=interpret,
    )


def kernel(input_ids, token_table, node_type_embeddings, label_nodes,
           label_edges, W1, b1, W2, b2):
    text_emb, mt = _make_sc()(input_ids, token_table, label_edges)
    return _make_tc()(text_emb, mt, label_nodes, node_type_embeddings,
                      W1, b1, W2, b2)


# speedup vs baseline: 25.2116x; 1.0010x over previous
import functools

import jax
import jax.numpy as jnp
from jax import lax
from jax.experimental import pallas as pl
from jax.experimental.pallas import tpu as pltpu
from jax.experimental.pallas import tpu_sc as plsc

NUM_LABELS = 1000
B = 32
SEQ = 128
HID = 768
DMODEL = 1024
TYPE_DIM = DMODEL - HID
E_LABEL = 4000

NC = 2
NS = 16
NW = NC * NS
LANES = 16

_TCH = 32
_CW = DMODEL // NW
_SRC_ROWS = NW * _CW
_DCOL = 1008
_NEG = E_LABEL // LANES
_NSLICE = HID // LANES
_RSEG = 16


def _sc_body(ids_hbm, tok_hbm, edges_hbm,
             text_out, mt_out,
             tcidx_a, tcidx_b, buf_a, buf_b, acc, esrc, edst, mt_loc,
             sem_a, sem_b):
    c = lax.axis_index("c")
    s = lax.axis_index("s")
    wid = s * NC + c
    zero16 = jnp.zeros((LANES,), jnp.float32)

    pltpu.sync_copy(ids_hbm.at[wid, pl.ds(0 * _TCH, _TCH)], tcidx_a)
    cp_a = pltpu.async_copy(tok_hbm.at[tcidx_a], buf_a, sem_a)
    pltpu.sync_copy(ids_hbm.at[wid, pl.ds(1 * _TCH, _TCH)], tcidx_b)
    cp_b = pltpu.async_copy(tok_hbm.at[tcidx_b], buf_b, sem_b)

    with jax.named_scope("ph_ecopy"):
        pltpu.sync_copy(edges_hbm.at[0], esrc)
        pltpu.sync_copy(edges_hbm.at[1], edst)

    def _zrow(j, carry):
        for t in range(_DCOL // LANES):
            mt_loc[j, pl.ds(t * LANES, LANES)] = zero16
        return carry
    with jax.named_scope("ph_zero"):
        lax.fori_loop(0, _CW, _zrow, 0)

    widv = lax.broadcast(wid, (LANES,))
    cwv = jnp.full((LANES,), _CW, jnp.int32)
    ones16 = jnp.ones((LANES,), jnp.float32)

    def _egroup(g, carry):
        gof = g * LANES
        sv = esrc[pl.ds(gof, LANES)]
        sblk = lax.div(sv, cwv)
        mine = sblk == widv
        nhit = plsc.all_reduce_population_count(mine)

        @pl.when(nhit[0] > 0)
        def _():
            dv = edst[pl.ds(gof, LANES)]
            slv = lax.rem(sv, cwv)
            val = jnp.where(mine, ones16, zero16)
            plsc.addupdate_scatter(mt_loc, [slv, dv], val)
        return carry

    with jax.named_scope("ph_scan1"):
        lax.fori_loop(0, _NEG // 2, _egroup, 0)
    with jax.named_scope("ph_waitA"):
        cp_a.wait()
    pltpu.sync_copy(ids_hbm.at[wid, pl.ds(2 * _TCH, _TCH)], tcidx_a)
    cp_a2 = pltpu.async_copy(tok_hbm.at[tcidx_a], buf_a, sem_a, add=True)
    with jax.named_scope("ph_scan2"):
        lax.fori_loop(_NEG // 2, _NEG, _egroup, 0)
    with jax.named_scope("ph_waitB"):
        cp_b.wait()
    pltpu.sync_copy(ids_hbm.at[wid, pl.ds(3 * _TCH, _TCH)], tcidx_b)
    cp_b2 = pltpu.async_copy(tok_hbm.at[tcidx_b], buf_b, sem_b, add=True)

    with jax.named_scope("ph_mtout"):
        pltpu.sync_copy(mt_loc, mt_out.at[pl.ds(wid * _CW, _CW)])
    with jax.named_scope("ph_waitA2B2"):
        cp_a2.wait()
        cp_b2.wait()

    inv = jnp.float32(1.0 / SEQ)
    _scope_red = jax.named_scope("ph_reduce")
    _scope_red.__enter__()
    for blk in range(_NSLICE // _RSEG):
        base = blk * _RSEG * LANES

        def _red(j, regs):
            return tuple(
                regs[t]
                + buf_a[j, pl.ds(base + t * LANES, LANES)]
                + buf_b[j, pl.ds(base + t * LANES, LANES)]
                for t in range(_RSEG))
        regs = lax.fori_loop(0, _TCH, _red, (zero16,) * _RSEG)
        for t in range(_RSEG):
            acc[pl.ds(base + t * LANES, LANES)] = regs[t] * inv
    _scope_red.__exit__(None, None, None)
    pltpu.sync_copy(acc, text_out.at[wid])


@functools.cache
def _make_sc(interpret=False):
    return pl.kernel(
        _sc_body,
        out_type=(
            jax.ShapeDtypeStruct((B, HID), jnp.float32),
            jax.ShapeDtypeStruct((_SRC_ROWS, _DCOL), jnp.float32),
        ),
        compiler_params=pltpu.CompilerParams(needs_layout_passes=False),
        mesh=plsc.VectorSubcoreMesh(core_axis_name="c", subcore_axis_name="s",
                                    num_cores=NC, num_subcores=NS),
        scratch_types=[
            pltpu.VMEM((_TCH,), jnp.int32),
            pltpu.VMEM((_TCH,), jnp.int32),
            pltpu.VMEM((_TCH, HID), jnp.float32),
            pltpu.VMEM((_TCH, HID), jnp.float32),
            pltpu.VMEM((HID,), jnp.float32),
            pltpu.VMEM((E_LABEL,), jnp.int32),
            pltpu.VMEM((E_LABEL,), jnp.int32),
            pltpu.VMEM((_CW, _DCOL), jnp.float32),
            pltpu.SemaphoreType.DMA,
            pltpu.SemaphoreType.DMA,
        ],
        interpret=interpret,
    )


def _tc_body(text_ref, mt_ref, lab_ref, nte_ref, w1_ref, b1_ref, w2_ref,
             b2_ref, out_ref):
    text = text_ref[...]
    nte1 = nte_ref[1, :]
    text_nodes = jnp.concatenate(
        [text, jnp.broadcast_to(nte1[None, :], (B, TYPE_DIM))], axis=1)
    s_text = jnp.sum(text_nodes, axis=0)
    lab = lab_ref[...]
    s_lab = jnp.sum(lab, axis=0)
    bf = jnp.bfloat16
    mt = mt_ref[...][:NUM_LABELS, :]
    seg_t = lax.dot_general(mt.astype(bf), lab.astype(bf),
                            (((0,), (0,)), ((), ())),
                            preferred_element_type=jnp.float32)
    seg = seg_t[:NUM_LABELS, :]
    agg = jnp.concatenate(
        [seg + s_text[None, :], text_nodes + s_lab[None, :]], axis=0)
    h = jnp.maximum(
        jnp.dot(agg.astype(bf), w1_ref[...].astype(bf),
                preferred_element_type=jnp.float32)
        + b1_ref[...][None, :], 0.0)
    out_ref[...] = (
        jnp.dot(h.astype(bf), w2_ref[...].astype(bf),
                preferred_element_type=jnp.float32)
        + b2_ref[...][None, :])


@functools.cache
def _make_tc(interpret=False):
    return pl.pallas_call(
        _tc_body,
        out_shape=jax.ShapeDtypeStruct((NUM_LABELS + B, NUM_LABELS),
                                       jnp.float32),
        interpret=interpret,
    )


def kernel(input_ids, token_table, node_type_embeddings, label_nodes,
           label_edges, W1, b1, W2, b2):
    text_emb, mt = _make_sc()(input_ids, token_table, label_edges)
    return _make_tc()(text_emb, mt, label_nodes, node_type_embeddings,
                      W1, b1, W2, b2)


# speedup vs baseline: 28.4009x; 1.1265x over previous
import functools

import jax
import jax.numpy as jnp
from jax import lax
from jax.experimental import pallas as pl
from jax.experimental.pallas import tpu as pltpu
from jax.experimental.pallas import tpu_sc as plsc

NUM_LABELS = 1000
B = 32
SEQ = 128
HID = 768
DMODEL = 1024
TYPE_DIM = DMODEL - HID
E_LABEL = 4000

NC = 2
NS = 16
NW = NC * NS
LANES = 16

_TCH = 32
_CW = DMODEL // NW
_SRC_ROWS = NW * _CW
_DCOL = 1008
_NEG = E_LABEL // LANES
_NSLICE = HID // LANES
_RSEG = 16


def _sc_body(ids_hbm, tok_hbm, edges_hbm,
             text_out, mt_out,
             tcidx_a, tcidx_b, buf_a, buf_b, acc, esrc, edst, mt_loc,
             sem_a, sem_b, sem_e):
    c = lax.axis_index("c")
    s = lax.axis_index("s")
    wid = s * NC + c
    zero16 = jnp.zeros((LANES,), jnp.float32)

    cp_src = pltpu.async_copy(edges_hbm.at[0], esrc, sem_e)
    cp_dst = pltpu.async_copy(edges_hbm.at[1], edst, sem_e)
    pltpu.sync_copy(ids_hbm.at[wid, pl.ds(0 * _TCH, _TCH)], tcidx_a)
    cp_a = pltpu.async_copy(tok_hbm.at[tcidx_a], buf_a, sem_a)
    pltpu.sync_copy(ids_hbm.at[wid, pl.ds(1 * _TCH, _TCH)], tcidx_b)
    cp_b = pltpu.async_copy(tok_hbm.at[tcidx_b], buf_b, sem_b)

    def _zrow(j, carry):
        for t in range(_DCOL // LANES):
            mt_loc[j, pl.ds(t * LANES, LANES)] = zero16
        return carry
    with jax.named_scope("ph_zero"):
        lax.fori_loop(0, _CW, _zrow, 0)
    with jax.named_scope("ph_ewait"):
        cp_src.wait()
        cp_dst.wait()

    widv = lax.broadcast(wid, (LANES,))
    five = jnp.full((LANES,), 5, jnp.int32)
    low5 = jnp.full((LANES,), _CW - 1, jnp.int32)
    ones16 = jnp.ones((LANES,), jnp.float32)

    def _egroup(g, carry):
        gof = g * LANES
        sv = esrc[pl.ds(gof, LANES)]
        dv = edst[pl.ds(gof, LANES)]
        sblk = lax.shift_right_logical(sv, five)
        slv = lax.bitwise_and(sv, low5)
        val = jnp.where(sblk == widv, ones16, zero16)
        plsc.addupdate_scatter(mt_loc, [slv, dv], val)
        return carry

    with jax.named_scope("ph_scan1"):
        lax.fori_loop(0, _NEG // 2, _egroup, 0)
    with jax.named_scope("ph_waitA"):
        cp_a.wait()
    pltpu.sync_copy(ids_hbm.at[wid, pl.ds(2 * _TCH, _TCH)], tcidx_a)
    cp_a2 = pltpu.async_copy(tok_hbm.at[tcidx_a], buf_a, sem_a, add=True)
    with jax.named_scope("ph_scan2"):
        lax.fori_loop(_NEG // 2, _NEG, _egroup, 0)
    with jax.named_scope("ph_waitB"):
        cp_b.wait()
    pltpu.sync_copy(ids_hbm.at[wid, pl.ds(3 * _TCH, _TCH)], tcidx_b)
    cp_b2 = pltpu.async_copy(tok_hbm.at[tcidx_b], buf_b, sem_b, add=True)

    cp_mt = pltpu.async_copy(mt_loc, mt_out.at[pl.ds(wid * _CW, _CW)], sem_e)
    with jax.named_scope("ph_waitA2B2"):
        cp_a2.wait()
        cp_b2.wait()

    inv = jnp.float32(1.0 / SEQ)
    _scope_red = jax.named_scope("ph_reduce")
    _scope_red.__enter__()
    for blk in range(_NSLICE // _RSEG):
        base = blk * _RSEG * LANES

        def _red(j, regs):
            return tuple(
                regs[t]
                + buf_a[j, pl.ds(base + t * LANES, LANES)]
                + buf_b[j, pl.ds(base + t * LANES, LANES)]
                for t in range(_RSEG))
        regs = lax.fori_loop(0, _TCH, _red, (zero16,) * _RSEG)
        for t in range(_RSEG):
            acc[pl.ds(base + t * LANES, LANES)] = regs[t] * inv
    _scope_red.__exit__(None, None, None)
    pltpu.sync_copy(acc, text_out.at[wid])
    with jax.named_scope("ph_mtwait"):
        cp_mt.wait()


@functools.cache
def _make_sc(interpret=False):
    return pl.kernel(
        _sc_body,
        out_type=(
            jax.ShapeDtypeStruct((B, HID), jnp.float32),
            jax.ShapeDtypeStruct((_SRC_ROWS, _DCOL), jnp.float32),
        ),
        compiler_params=pltpu.CompilerParams(needs_layout_passes=False),
        mesh=plsc.VectorSubcoreMesh(core_axis_name="c", subcore_axis_name="s",
                                    num_cores=NC, num_subcores=NS),
        scratch_types=[
            pltpu.VMEM((_TCH,), jnp.int32),
            pltpu.VMEM((_TCH,), jnp.int32),
            pltpu.VMEM((_TCH, HID), jnp.float32),
            pltpu.VMEM((_TCH, HID), jnp.float32),
            pltpu.VMEM((HID,), jnp.float32),
            pltpu.VMEM((E_LABEL,), jnp.int32),
            pltpu.VMEM((E_LABEL,), jnp.int32),
            pltpu.VMEM((_CW, _DCOL), jnp.float32),
            pltpu.SemaphoreType.DMA,
            pltpu.SemaphoreType.DMA,
            pltpu.SemaphoreType.DMA,
        ],
        interpret=interpret,
    )


def _tc_body(text_ref, mt_ref, lab_ref, nte_ref, w1_ref, b1_ref, w2_ref,
             b2_ref, out_ref):
    text = text_ref[...]
    nte1 = nte_ref[1, :]
    text_nodes = jnp.concatenate(
        [text, jnp.broadcast_to(nte1[None, :], (B, TYPE_DIM))], axis=1)
    s_text = jnp.sum(text_nodes, axis=0)
    lab = lab_ref[...]
    s_lab = jnp.sum(lab, axis=0)
    bf = jnp.bfloat16
    mt = mt_ref[...][:NUM_LABELS, :]
    seg_t = lax.dot_general(mt.astype(bf), lab.astype(bf),
                            (((0,), (0,)), ((), ())),
                            preferred_element_type=jnp.float32)
    seg = seg_t[:NUM_LABELS, :]
    w1b = w1_ref[...].astype(bf)
    w2b = w2_ref[...].astype(bf)
    b1v = b1_ref[...][None, :]
    b2v = b2_ref[...][None, :]
    h_lab = jnp.maximum(
        jnp.dot((seg + s_text[None, :]).astype(bf), w1b,
                preferred_element_type=jnp.float32) + b1v, 0.0)
    h_txt = jnp.maximum(
        jnp.dot((text_nodes + s_lab[None, :]).astype(bf), w1b,
                preferred_element_type=jnp.float32) + b1v, 0.0)
    out_ref[pl.ds(0, NUM_LABELS), :] = (
        jnp.dot(h_lab.astype(bf), w2b,
                preferred_element_type=jnp.float32) + b2v)
    out_ref[pl.ds(NUM_LABELS, B), :] = (
        jnp.dot(h_txt.astype(bf), w2b,
                preferred_element_type=jnp.float32) + b2v)


@functools.cache
def _make_tc(interpret=False):
    return pl.pallas_call(
        _tc_body,
        out_shape=jax.ShapeDtypeStruct((NUM_LABELS + B, NUM_LABELS),
                                       jnp.float32),
        interpret=interpret,
    )


def kernel(input_ids, token_table, node_type_embeddings, label_nodes,
           label_edges, W1, b1, W2, b2):
    text_emb, mt = _make_sc()(input_ids, token_table, label_edges)
    return _make_tc()(text_emb, mt, label_nodes, node_type_embeddings,
                      W1, b1, W2, b2)
